# baseline (speedup 1.0000x reference)
import jax
import jax.numpy as jnp
from jax import lax
from jax.experimental import pallas as pl
from jax.experimental.pallas import tpu as pltpu
from jax.experimental.pallas import tpu_sc as plsc

_NB = 4
_NC = 768
_D = 8192
_ROWS = _NB * _NC
_NW = 32
_RPW = _ROWS // _NW
_K = 6
_NCHUNK = _RPW // _K


def _body(in_hbm, out_hbm, buf0, buf1, sem_ld, sem_st):
    cid = lax.axis_index("c")
    sid = lax.axis_index("s")
    wid = cid * 16 + sid
    base = wid * _RPW
    b = base // _NC
    src_hi = 2 * b * _NC + (_NC - 1) - base

    bufs = (buf0, buf1)

    def start_load(g):
        src_lo = src_hi - g * _K - (_K - 1)
        return pltpu.async_copy(
            in_hbm.at[pl.ds(src_lo * _D, _K * _D)], bufs[g % 2], sem_ld)

    def start_stores(g):
        cps = []
        for t in range(_K):
            dst = base + g * _K + t
            cps.append(pltpu.async_copy(
                bufs[g % 2].at[pl.ds((_K - 1 - t) * _D, _D)],
                out_hbm.at[pl.ds(dst * _D, _D)], sem_st))
        return cps

    prev_stores = None
    cur_ld = start_load(0)
    for g in range(_NCHUNK):
        if prev_stores is not None:
            for cp in prev_stores:
                cp.wait()
        nxt_ld = start_load(g + 1) if g + 1 < _NCHUNK else None
        cur_ld.wait()
        prev_stores = start_stores(g)
        cur_ld = nxt_ld
    for cp in prev_stores:
        cp.wait()


@jax.jit
def _permute(x1d):
    mesh = plsc.VectorSubcoreMesh(core_axis_name="c", subcore_axis_name="s")
    return pl.kernel(
        _body,
        mesh=mesh,
        out_type=jax.ShapeDtypeStruct((_ROWS * _D,), jnp.float32),
        scratch_types=[
            pltpu.VMEM((_K * _D,), jnp.float32),
            pltpu.VMEM((_K * _D,), jnp.float32),
            pltpu.SemaphoreType.DMA,
            pltpu.SemaphoreType.DMA,
        ],
    )(x1d)


def kernel(input):
    x1d = input.reshape(_ROWS * _D)
    out = _permute(x1d)
    return out.reshape(_NB, _NC, _D)
